# Initial kernel scaffold; baseline (speedup 1.0000x reference)
import functools

import jax
import jax.numpy as jnp
from jax import lax
from jax.experimental import pallas as pl
from jax.experimental.pallas import tpu as pltpu
from jax.experimental.pallas import tpu_sc as plsc

_B = 16384
_V = 1000

_info = plsc.get_sparse_core_info()
_NC = _info.num_cores
_NS = _info.num_subcores
_L = _info.num_lanes
_NW = _NC * _NS
_PW = _B // _NW
_NVEC = _PW // _L
_CHUNK = 128
_NCHUNK = _PW // _CHUNK
_VPC = _CHUNK // _L


def _body(preds_hbm, labels_hbm, lcas_hbm, cov_hbm, out_hbm,
          preds_v, labels_v, idx_v, lca_v, cov_v, res_v, sem):
    wid = lax.axis_index("s") * _NC + lax.axis_index("c")
    base = wid * _PW

    pltpu.sync_copy(preds_hbm.at[pl.ds(base, _PW)], preds_v)
    pltpu.sync_copy(labels_hbm.at[pl.ds(base, _PW)], labels_v)
    pltpu.sync_copy(cov_hbm, cov_v)

    for s in range(_NVEC):
        p16 = preds_v[pl.ds(s * _L, _L)]
        l16 = labels_v[pl.ds(s * _L, _L)]
        idx_v[s // _VPC, pl.ds((s % _VPC) * _L, _L)] = p16 * _V + l16

    copies = [
        pltpu.async_copy(lcas_hbm.at[idx_v.at[j]], lca_v.at[j], sem)
        for j in range(_NCHUNK)
    ]
    for c in copies:
        c.wait()

    acc = jnp.zeros((_L,), jnp.float32)
    one = jnp.full((_L,), 1.0, jnp.float32)
    zero = jnp.zeros((_L,), jnp.float32)
    for s in range(_NVEC):
        lca16 = lca_v[s // _VPC, pl.ds((s % _VPC) * _L, _L)]
        p16 = preds_v[pl.ds(s * _L, _L)]
        lca_cov = plsc.load_gather(cov_v, [lca16])
        pred_cov = plsc.load_gather(cov_v, [p16])
        rel = jnp.where(pred_cov != zero, lca_cov / pred_cov, one)
        acc = acc + (one - rel)

    total = lax.reduce_sum_p.bind(acc, axes=(0,))
    res_v[...] = jnp.broadcast_to(total, (_L,))
    pltpu.sync_copy(res_v, out_hbm.at[wid])


_sc_call = functools.partial(
    pl.kernel,
    out_type=jax.ShapeDtypeStruct((_NW, _L), jnp.float32),
    mesh=plsc.VectorSubcoreMesh(core_axis_name="c", subcore_axis_name="s"),
    scratch_types=[
        pltpu.VMEM((_PW,), jnp.int32),
        pltpu.VMEM((_PW,), jnp.int32),
        pltpu.VMEM((_NCHUNK, _CHUNK), jnp.int32),
        pltpu.VMEM((_NCHUNK, _CHUNK), jnp.int32),
        pltpu.VMEM((_V,), jnp.float32),
        pltpu.VMEM((_L,), jnp.float32),
        pltpu.SemaphoreType.DMA,
    ],
)(_body)


def kernel(preds, labels, lcas, coverage_vec):
    preds32 = preds.astype(jnp.int32)
    labels32 = labels.astype(jnp.int32)
    lcas_flat = lcas.astype(jnp.int32).reshape(_V * _V)
    cov_flat = coverage_vec.reshape(_V)
    partials = _sc_call(preds32, labels32, lcas_flat, cov_flat)
    return jnp.sum(partials[:, 0]) / _B


# speedup vs baseline: 4.4277x; 4.4277x over previous
import functools

import jax
import jax.numpy as jnp
from jax import lax
from jax.experimental import pallas as pl
from jax.experimental.pallas import tpu as pltpu
from jax.experimental.pallas import tpu_sc as plsc

_B = 16384
_V = 1000

_info = plsc.get_sparse_core_info()
_NC = _info.num_cores
_NS = _info.num_subcores
_L = _info.num_lanes
_NW = _NC * _NS
_PW = _B // _NW
_NVEC = _PW // _L
_CHUNK = 128
_NCHUNK = _PW // _CHUNK
_VPC = _CHUNK // _L


def _body(preds_hbm, labels_hbm, lcas_hbm, cov_hbm, out_hbm,
          preds_v, labels_v, idx_v, lca_v, pcov_v, lcov_v, res_v,
          sem_a, sem_b):
    wid = lax.axis_index("s") * _NC + lax.axis_index("c")
    row0 = wid * _NCHUNK

    pltpu.sync_copy(preds_hbm.at[pl.ds(row0, _NCHUNK)], preds_v)
    pltpu.sync_copy(labels_hbm.at[pl.ds(row0, _NCHUNK)], labels_v)

    for j in range(_NCHUNK):
        for v in range(_VPC):
            p16 = preds_v[j, pl.ds(v * _L, _L)]
            l16 = labels_v[j, pl.ds(v * _L, _L)]
            idx_v[j, pl.ds(v * _L, _L)] = p16 * _V + l16

    lca_copies = [
        pltpu.async_copy(lcas_hbm.at[idx_v.at[j]], lca_v.at[j], sem_a)
        for j in range(_NCHUNK)
    ]
    pcov_copies = [
        pltpu.async_copy(cov_hbm.at[preds_v.at[j]], pcov_v.at[j], sem_b)
        for j in range(_NCHUNK)
    ]
    lcov_copies = []
    for j in range(_NCHUNK):
        lca_copies[j].wait()
        lcov_copies.append(
            pltpu.async_copy(cov_hbm.at[lca_v.at[j]], lcov_v.at[j], sem_b))
    for c in pcov_copies:
        c.wait()
    for c in lcov_copies:
        c.wait()

    acc = jnp.zeros((_L,), jnp.float32)
    one = jnp.full((_L,), 1.0, jnp.float32)
    zero = jnp.zeros((_L,), jnp.float32)
    for j in range(_NCHUNK):
        for v in range(_VPC):
            lc = lcov_v[j, pl.ds(v * _L, _L)]
            pc = pcov_v[j, pl.ds(v * _L, _L)]
            rel = jnp.where(pc != zero, lc / pc, one)
            acc = acc + (one - rel)

    res_v[...] = acc
    pltpu.sync_copy(res_v, out_hbm.at[wid])


_sc_call = functools.partial(
    pl.kernel,
    out_type=jax.ShapeDtypeStruct((_NW, _L), jnp.float32),
    mesh=plsc.VectorSubcoreMesh(core_axis_name="c", subcore_axis_name="s"),
    scratch_types=[
        pltpu.VMEM((_NCHUNK, _CHUNK), jnp.int32),
        pltpu.VMEM((_NCHUNK, _CHUNK), jnp.int32),
        pltpu.VMEM((_NCHUNK, _CHUNK), jnp.int32),
        pltpu.VMEM((_NCHUNK, _CHUNK), jnp.int32),
        pltpu.VMEM((_NCHUNK, _CHUNK), jnp.float32),
        pltpu.VMEM((_NCHUNK, _CHUNK), jnp.float32),
        pltpu.VMEM((_L,), jnp.float32),
        pltpu.SemaphoreType.DMA,
        pltpu.SemaphoreType.DMA,
    ],
)(_body)


def kernel(preds, labels, lcas, coverage_vec):
    preds2d = preds.astype(jnp.int32).reshape(_B // _CHUNK, _CHUNK)
    labels2d = labels.astype(jnp.int32).reshape(_B // _CHUNK, _CHUNK)
    lcas_flat = lcas.astype(jnp.int32).reshape(_V * _V)
    cov_flat = coverage_vec.reshape(_V)
    partials = _sc_call(preds2d, labels2d, lcas_flat, cov_flat)
    return jnp.sum(partials) / _B


# speedup vs baseline: 7.1442x; 1.6135x over previous
import functools

import jax
import jax.numpy as jnp
from jax import lax
from jax.experimental import pallas as pl
from jax.experimental.pallas import tpu as pltpu
from jax.experimental.pallas import tpu_sc as plsc

_B = 16384
_V = 1000

_info = plsc.get_sparse_core_info()
_NC = _info.num_cores
_NS = _info.num_subcores
_L = _info.num_lanes
_NW = _NC * _NS
_PW = _B // _NW
_CHUNK = 128
_NCHUNK = _PW // _CHUNK
_VPC = _CHUNK // _L


def _body(preds_hbm, labels_hbm, lcas_hbm, cov_hbm, out_hbm,
          preds_v, labels_v, idx_v, lca_v, cov_v, res_v, sem):
    wid = lax.axis_index("s") * _NC + lax.axis_index("c")
    row0 = wid * _NCHUNK

    pltpu.sync_copy(preds_hbm.at[pl.ds(row0, _NCHUNK)], preds_v)
    pltpu.sync_copy(labels_hbm.at[pl.ds(row0, _NCHUNK)], labels_v)
    pltpu.sync_copy(cov_hbm, cov_v)

    for j in range(_NCHUNK):
        for v in range(_VPC):
            p16 = preds_v[j, pl.ds(v * _L, _L)]
            l16 = labels_v[j, pl.ds(v * _L, _L)]
            idx_v[j, pl.ds(v * _L, _L)] = p16 * _V + l16

    copies = [
        pltpu.async_copy(lcas_hbm.at[idx_v.at[j]], lca_v.at[j], sem)
        for j in range(_NCHUNK)
    ]
    for c in copies:
        c.wait()

    acc = jnp.zeros((_L,), jnp.float32)
    one = jnp.full((_L,), 1.0, jnp.float32)
    zero = jnp.zeros((_L,), jnp.float32)
    for j in range(_NCHUNK):
        for v in range(_VPC):
            lca16 = lca_v[j, pl.ds(v * _L, _L)]
            p16 = preds_v[j, pl.ds(v * _L, _L)]
            lca_cov = plsc.load_gather(cov_v, [lca16])
            pred_cov = plsc.load_gather(cov_v, [p16])
            rel = jnp.where(pred_cov != zero, lca_cov / pred_cov, one)
            acc = acc + (one - rel)

    total = lax.reduce_sum_p.bind(acc, axes=(0,))
    res_v[...] = jnp.broadcast_to(total, (_L,))
    pltpu.sync_copy(res_v, out_hbm.at[wid])


_sc_call = functools.partial(
    pl.kernel,
    out_type=jax.ShapeDtypeStruct((_NW, _L), jnp.float32),
    mesh=plsc.VectorSubcoreMesh(core_axis_name="c", subcore_axis_name="s"),
    compiler_params=pltpu.CompilerParams(needs_layout_passes=False),
    scratch_types=[
        pltpu.VMEM((_NCHUNK, _CHUNK), jnp.int32),
        pltpu.VMEM((_NCHUNK, _CHUNK), jnp.int32),
        pltpu.VMEM((_NCHUNK, _CHUNK), jnp.int32),
        pltpu.VMEM((_NCHUNK, _CHUNK), jnp.int32),
        pltpu.VMEM((_V,), jnp.float32),
        pltpu.VMEM((_L,), jnp.float32),
        pltpu.SemaphoreType.DMA,
    ],
)(_body)


def kernel(preds, labels, lcas, coverage_vec):
    preds2d = preds.astype(jnp.int32).reshape(_B // _CHUNK, _CHUNK)
    labels2d = labels.astype(jnp.int32).reshape(_B // _CHUNK, _CHUNK)
    lcas_flat = lcas.astype(jnp.int32).reshape(_V * _V)
    cov_flat = coverage_vec.reshape(_V)
    partials = _sc_call(preds2d, labels2d, lcas_flat, cov_flat)
    return jnp.sum(partials[:, 0]) / _B
